# Initial kernel scaffold; baseline (speedup 1.0000x reference)
import jax
import jax.numpy as jnp
from jax.experimental import pallas as pl


def kernel(x):
    raise NotImplementedError("write your pallas kernel here")


# speedup vs baseline: 7.6561x; 7.6561x over previous
import jax
import jax.numpy as jnp
from jax import lax
from jax.experimental import pallas as pl
from jax.experimental.pallas import tpu as pltpu

CONF_THRES = 0.25
IOU_THRES = 0.45
MAX_DET = 1000
MAX_WH = 4096.0

N_RAW = 20000
N_PAD = 20480
ROWS = 160
PREP_CHUNK = 16
NCLS = 80
BIG = 1.0e9


def _prep_kernel(x_ref, s_ref, x1_ref, y1_ref, x2_ref, y2_ref, a_ref, j_ref,
                 ux1_ref, uy1_ref, ux2_ref, uy2_ref):
    blk = x_ref[0]
    cls = blk[:, 0:NCLS, :]
    obj = blk[:, NCLS, :]
    sc = cls * obj[:, None, :]
    conf = jnp.max(sc, axis=1)
    ii = lax.broadcasted_iota(jnp.int32, (PREP_CHUNK, NCLS, 128), 1).astype(jnp.float32)
    jf = jnp.min(jnp.where(sc == conf[:, None, :], ii, 1e9), axis=1)
    valid = (obj > CONF_THRES) & (conf > CONF_THRES)
    score = jnp.where(valid, conf, -1.0)

    cx = blk[:, NCLS + 1, :]
    cy = blk[:, NCLS + 2, :]
    w = blk[:, NCLS + 3, :]
    h = blk[:, NCLS + 4, :]
    off = jf * MAX_WH
    hx = w * 0.5
    hy = h * 0.5
    x1 = cx - hx
    y1 = cy - hy
    x2 = cx + hx
    y2 = cy + hy
    ox1 = x1 + off
    oy1 = y1 + off
    ox2 = x2 + off
    oy2 = y2 + off
    s_ref[0] = score
    x1_ref[0] = ox1
    y1_ref[0] = oy1
    x2_ref[0] = ox2
    y2_ref[0] = oy2
    a_ref[0] = (ox2 - ox1) * (oy2 - oy1)
    j_ref[0] = jf
    ux1_ref[0] = x1
    uy1_ref[0] = y1
    ux2_ref[0] = x2
    uy2_ref[0] = y2


def _nms_kernel(s_in, x1_ref, y1_ref, x2_ref, y2_ref, a_ref, j_ref,
                ux1_ref, uy1_ref, ux2_ref, uy2_ref, out_ref, s_ref, kx1_ref, ky1_ref, kx2_ref, ky2_ref, ka_ref,
                m_sc):
    s_ref[...] = s_in[0]
    kx1_ref[...] = jnp.full((NCLS, 128), BIG, jnp.float32)
    ky1_ref[...] = jnp.full((NCLS, 128), BIG, jnp.float32)
    kx2_ref[...] = jnp.full((NCLS, 128), -BIG, jnp.float32)
    ky2_ref[...] = jnp.full((NCLS, 128), -BIG, jnp.float32)
    ka_ref[...] = jnp.zeros((NCLS, 128), jnp.float32)

    riota = lax.broadcasted_iota(jnp.int32, (ROWS, 128), 0).astype(jnp.float32)
    liota = lax.broadcasted_iota(jnp.int32, (ROWS, 128), 1).astype(jnp.float32)
    fiota = riota * 128.0 + liota
    oiota = lax.broadcasted_iota(jnp.int32, (8, 1024), 1)
    lane1 = lax.broadcasted_iota(jnp.int32, (1, 128), 1)

    def body(carry):
        k, m, acc = carry
        s = s_ref[...]
        p = jnp.min(jnp.where(s == m, fiota, 3.0e7))
        pi = p.astype(jnp.int32)
        r = pi // 128
        c = pi - r * 128

        row = s_ref[pl.ds(r, 1), :]
        s_ref[pl.ds(r, 1), :] = jnp.where(lane1 == c, -1.0, row)
        m_sc[0] = jnp.max(jnp.where(fiota == p, -1.0, s))

        onehot = (lane1 == c).astype(jnp.float32)

        def pick(ref):
            return jnp.sum(ref[0, pl.ds(r, 1), :] * onehot)

        bx1 = pick(x1_ref)
        by1 = pick(y1_ref)
        bx2 = pick(x2_ref)
        by2 = pick(y2_ref)
        ba = pick(a_ref)
        bj = pick(j_ref)
        ci = bj.astype(jnp.int32)

        r1 = kx1_ref[pl.ds(ci, 1), :]
        r2 = ky1_ref[pl.ds(ci, 1), :]
        r3 = kx2_ref[pl.ds(ci, 1), :]
        r4 = ky2_ref[pl.ds(ci, 1), :]
        r5 = ka_ref[pl.ds(ci, 1), :]
        iw = jnp.maximum(jnp.minimum(bx2, r3) - jnp.maximum(bx1, r1), 0.0)
        ih = jnp.maximum(jnp.minimum(by2, r4) - jnp.maximum(by1, r2), 0.0)
        inter = iw * ih
        iou = inter / (r5 + ba - inter + 1e-9)
        keep = jnp.logical_not(jnp.any(iou > IOU_THRES))

        kc = jnp.sum((r3 > -0.5 * BIG).astype(jnp.int32))
        tgt = jnp.where(keep, kc, -1)
        put = lane1 == tgt
        kx1_ref[pl.ds(ci, 1), :] = jnp.where(put, bx1, r1)
        ky1_ref[pl.ds(ci, 1), :] = jnp.where(put, by1, r2)
        kx2_ref[pl.ds(ci, 1), :] = jnp.where(put, bx2, r3)
        ky2_ref[pl.ds(ci, 1), :] = jnp.where(put, by2, r4)
        ka_ref[pl.ds(ci, 1), :] = jnp.where(put, ba, r5)

        @pl.when(keep & (kc >= 128))
        def _():
            s2 = s_ref[...]
            iw2 = jnp.maximum(
                jnp.minimum(bx2, x2_ref[0]) - jnp.maximum(bx1, x1_ref[0]), 0.0)
            ih2 = jnp.maximum(
                jnp.minimum(by2, y2_ref[0]) - jnp.maximum(by1, y1_ref[0]), 0.0)
            inter2 = iw2 * ih2
            iou2 = inter2 / (ba + a_ref[0] - inter2 + 1e-9)
            s3 = jnp.where(iou2 > IOU_THRES, -1.0, s2)
            s_ref[...] = s3
            m_sc[0] = jnp.max(s3)

        vals = jnp.concatenate([
            jnp.full((1, 1), pick(ux1_ref), jnp.float32),
            jnp.full((1, 1), pick(uy1_ref), jnp.float32),
            jnp.full((1, 1), pick(ux2_ref), jnp.float32),
            jnp.full((1, 1), pick(uy2_ref), jnp.float32),
            jnp.full((1, 1), m, jnp.float32),
            jnp.full((1, 1), bj, jnp.float32),
            jnp.zeros((2, 1), jnp.float32),
        ], axis=0)
        ktgt = jnp.where(keep, k, -1)
        acc = jnp.where(oiota == ktgt, vals, acc)
        return k + keep.astype(jnp.int32), m_sc[0], acc

    def cond(carry):
        k, m, _ = carry
        return (k < MAX_DET) & (m > 0.0)

    m0 = jnp.max(s_ref[...])
    acc0 = jnp.zeros((8, 1024), jnp.float32)
    _, _, acc = lax.while_loop(cond, body, (jnp.int32(0), m0, acc0))
    out_ref[0] = acc


def kernel(x):
    p = x[0]
    B = p.shape[0]
    feats = jnp.concatenate([
        p[:, :, 5:85],
        p[:, :, 4:5],
        p[:, :, 0:4],
        jnp.zeros((B, N_RAW, 3), jnp.float32),
    ], axis=-1)
    feats = jnp.pad(feats, ((0, 0), (0, N_PAD - N_RAW), (0, 0)))
    xt = feats.reshape(B, ROWS, 128, 88).transpose(0, 1, 3, 2)

    comp_shape = jax.ShapeDtypeStruct((B, ROWS, 128), jnp.float32)
    comps = pl.pallas_call(
        _prep_kernel,
        grid=(B, ROWS // PREP_CHUNK),
        in_specs=[pl.BlockSpec((1, PREP_CHUNK, 88, 128),
                               lambda b, c: (b, c, 0, 0))],
        out_specs=[pl.BlockSpec((1, PREP_CHUNK, 128), lambda b, c: (b, c, 0))
                   ] * 11,
        out_shape=[comp_shape] * 11,
    )(xt)
    s, x1, y1, x2, y2, a, j, ux1, uy1, ux2, uy2 = comps

    full = pl.BlockSpec((1, ROWS, 128), lambda b: (b, 0, 0))
    out = pl.pallas_call(
        _nms_kernel,
        grid=(B,),
        in_specs=[full] * 11,
        out_specs=pl.BlockSpec((1, 8, 1024), lambda b: (b, 0, 0)),
        out_shape=jax.ShapeDtypeStruct((B, 8, 1024), jnp.float32),
        scratch_shapes=[pltpu.VMEM((ROWS, 128), jnp.float32)] +
                       [pltpu.VMEM((NCLS, 128), jnp.float32)] * 5 +
                       [pltpu.SMEM((1,), jnp.float32)],
    )(s, x1, y1, x2, y2, a, j, ux1, uy1, ux2, uy2)

    return out.transpose(0, 2, 1)[:, :MAX_DET, :6]
